# baseline (speedup 1.0000x reference)
import functools

import jax
import jax.numpy as jnp
from jax.experimental import pallas as pl
from jax.experimental.pallas import tpu as pltpu
from jax.experimental.pallas import tpu_sc as plsc

N = 10000
E = 320000
HD = 128
EHD = 64
BE = 2560
GE = E // BE
CHUNK = 128
NCHUNK = E // CHUNK
NSUB = 16
NPSA = 624
NTAIL = N - NSUB * NPSA

_SC_MESH = plsc.VectorSubcoreMesh(core_axis_name="c", subcore_axis_name="s")


def _ln(x, g, b):
    m = jnp.mean(x, axis=-1, keepdims=True)
    v = jnp.mean((x - m) * (x - m), axis=-1, keepdims=True)
    return (x - m) / jnp.sqrt(v + 1e-5) * g + b


def _dot(a, b):
    return jnp.dot(a.astype(jnp.bfloat16), b.astype(jnp.bfloat16),
                   preferred_element_type=jnp.float32)


def _ef_in_body(pvs_ref, pvd_ref, o_ref):
    a = pvs_ref[...]
    b = pvd_ref[...]
    rel_pos = a[:, 0:3] - b[:, 0:3]
    rel_vel = a[:, 3:6] - b[:, 3:6]
    dist = jnp.sqrt(jnp.sum(rel_pos * rel_pos, axis=-1, keepdims=True))
    speed = jnp.sqrt(jnp.sum(rel_vel * rel_vel, axis=-1, keepdims=True))
    o_ref[...] = jnp.concatenate([rel_pos, dist, rel_vel, speed], axis=-1)


def _ef_in(pvs, pvd):
    return pl.pallas_call(
        _ef_in_body,
        grid=(GE,),
        in_specs=[pl.BlockSpec((BE, 16), lambda i: (i, 0)),
                  pl.BlockSpec((BE, 16), lambda i: (i, 0))],
        out_specs=pl.BlockSpec((BE, 8), lambda i: (i, 0)),
        out_shape=jax.ShapeDtypeStruct((E, 8), jnp.float32),
    )(pvs, pvd)


def _enc_body(ns_ref, w1_ref, b1_ref, g_ref, bb_ref, w2_ref, b2_ref, o_ref):
    h = _dot(ns_ref[...], w1_ref[...]) + b1_ref[...]
    h = jnp.maximum(_ln(h, g_ref[...], bb_ref[...]), 0.0)
    o_ref[...] = _dot(h, w2_ref[...]) + b2_ref[...]


def _encoder(ns, p):
    return pl.pallas_call(
        _enc_body,
        out_shape=jax.ShapeDtypeStruct((N, HD), jnp.float32),
    )(ns, p['ne_w1'], p['ne_b1'], p['ne_ln_g'], p['ne_ln_b'], p['ne_w2'], p['ne_b2'])


def _tables(x, lp):
    wi = lp['msg_w1'][EHD:EHD + HD]
    wj = lp['msg_w1'][EHD + HD:]

    def body(x_ref, wi_ref, wj_ref, b1_ref, pi_ref, pj_ref):
        x = x_ref[...]
        pi_ref[...] = _dot(x, wi_ref[...]) + b1_ref[...].reshape(1, HD)
        pj_ref[...] = _dot(x, wj_ref[...])

    return pl.pallas_call(
        body,
        out_shape=(jax.ShapeDtypeStruct((N, HD), jnp.float32),
                   jax.ShapeDtypeStruct((N, HD), jnp.float32)),
    )(x, wi, wj, lp['msg_b1'])


def _edge_body(ef_ref, gi_ref, gj_ref, w1_ref, b1_ref, w2_ref, b2_ref,
               we_ref, g_ref, b_ref, mw2_ref, mb2_ref, o_ref):
    t = jnp.maximum(_dot(ef_ref[...], w1_ref[...]) + b1_ref[...], 0.0)
    e = _dot(t, w2_ref[...]) + b2_ref[...]
    m1 = _dot(e, we_ref[...]) + gi_ref[...] + gj_ref[...]
    h = jnp.maximum(_ln(m1, g_ref[...], b_ref[...]), 0.0)
    o_ref[...] = _dot(h, mw2_ref[...]) + mb2_ref[...]


def _edge_stage(ef_in, gi, gj, p, lp):
    we = lp['msg_w1'][0:EHD]
    return pl.pallas_call(
        _edge_body,
        grid=(GE,),
        in_specs=[pl.BlockSpec((BE, 8), lambda i: (i, 0)),
                  pl.BlockSpec((BE, HD), lambda i: (i, 0)),
                  pl.BlockSpec((BE, HD), lambda i: (i, 0)),
                  pl.BlockSpec((8, EHD), lambda i: (0, 0)),
                  pl.BlockSpec((EHD,), lambda i: (0,)),
                  pl.BlockSpec((EHD, EHD), lambda i: (0, 0)),
                  pl.BlockSpec((EHD,), lambda i: (0,)),
                  pl.BlockSpec((EHD, HD), lambda i: (0, 0)),
                  pl.BlockSpec((HD,), lambda i: (0,)),
                  pl.BlockSpec((HD,), lambda i: (0,)),
                  pl.BlockSpec((HD, HD), lambda i: (0, 0)),
                  pl.BlockSpec((HD,), lambda i: (0,))],
        out_specs=pl.BlockSpec((BE, HD), lambda i: (i, 0)),
        out_shape=jax.ShapeDtypeStruct((E, HD), jnp.float32),
    )(ef_in, gi, gj, p['ef_w1'], p['ef_b1'], p['ef_w2'], p['ef_b2'],
      we, lp['msg_ln_g'], lp['msg_ln_b'], lp['msg_w2'], lp['msg_b2'])


def _node_body(pa_ref, pb_ref, x_ref,
               uw1a_ref, uw1b_ref, ub1_ref, ug_ref, ubb_ref, uw2_ref, ub2_ref,
               o_ref):
    aggr = pa_ref[...] + pb_ref[...]
    x = x_ref[...]
    ui = (_dot(x, uw1a_ref[...]) + _dot(aggr, uw1b_ref[...]) + ub1_ref[...])
    u = jnp.maximum(_ln(ui, ug_ref[...], ubb_ref[...]), 0.0)
    o_ref[...] = _dot(u, uw2_ref[...]) + ub2_ref[...] + x


def _node_stage(pa, pb, x, lp):
    return pl.pallas_call(
        _node_body,
        out_shape=jax.ShapeDtypeStruct((N, HD), jnp.float32),
    )(pa, pb, x,
      lp['upd_w1'][0:HD], lp['upd_w1'][HD:], lp['upd_b1'],
      lp['upd_ln_g'], lp['upd_ln_b'], lp['upd_w2'], lp['upd_b2'])


def _dec_body(x_ref, w1_ref, b1_ref, g_ref, bb_ref, w2_ref, b2_ref, w3_ref,
              b3_ref, o_ref):
    d = _dot(x_ref[...], w1_ref[...]) + b1_ref[...]
    d = jnp.maximum(_ln(d, g_ref[...], bb_ref[...]), 0.0)
    d = jnp.maximum(_dot(d, w2_ref[...]) + b2_ref[...], 0.0)
    o_ref[...] = _dot(d, w3_ref[...]) + b3_ref[...]


def _decoder(x, p):
    return pl.pallas_call(
        _dec_body,
        out_shape=jax.ShapeDtypeStruct((N, 3), jnp.float32),
    )(x, p['dec_w1'], p['dec_b1'], p['dec_ln_g'], p['dec_ln_b'],
      p['dec_w2'], p['dec_b2'], p['dec_w3'], p['dec_b3'])


def _sc_gather2(ti, tj, idx_i, idx_j, tc_tiling=True):
    D = ti.shape[1]

    @functools.partial(
        pl.kernel,
        out_type=(jax.ShapeDtypeStruct((E, D), jnp.float32),
                  jax.ShapeDtypeStruct((E, D), jnp.float32)),
        mesh=_SC_MESH,
        compiler_params=pltpu.CompilerParams(use_tc_tiling_on_sc=tc_tiling),
    )
    def k(ti_hbm, tj_hbm, ii_hbm, ij_hbm, gi_hbm, gj_hbm):
        def body(ii_v, ij_v, gi_v, gj_v):
            pltpu.sync_copy(ti_hbm.at[ii_v.at[0]], gi_v)
            pltpu.sync_copy(tj_hbm.at[ij_v.at[0]], gj_v)

        pltpu.emit_pipeline(
            body,
            grid=(NCHUNK,),
            in_specs=[pl.BlockSpec((1, CHUNK), lambda i: (0, i)),
                      pl.BlockSpec((1, CHUNK), lambda i: (0, i))],
            out_specs=[pl.BlockSpec((CHUNK, D), lambda i: (i, 0)),
                       pl.BlockSpec((CHUNK, D), lambda i: (i, 0))],
            core_axis_name=("c", "s"),
            dimension_semantics=(pltpu.PARALLEL,),
        )(ii_hbm, ij_hbm, gi_hbm, gj_hbm)

    return k(ti, tj, idx_i.reshape(1, E), idx_j.reshape(1, E))


def _sc_scatter_add(m, dst):
    D = m.shape[1]
    zeros = jnp.zeros((NPSA, D), jnp.float32)

    @functools.partial(
        pl.kernel,
        out_type=jax.ShapeDtypeStruct((2, N, D), jnp.float32),
        mesh=_SC_MESH,
        scratch_types=[pltpu.VMEM_SHARED((N, D), jnp.float32)],
    )
    def k(m_hbm, di_hbm, z_hbm, out_hbm, acc_sh):
        ci = jax.lax.axis_index("c")
        si = jax.lax.axis_index("s")
        pltpu.sync_copy(z_hbm, acc_sh.at[pl.ds(si * NPSA, NPSA)])

        @pl.when(si == 0)
        def _():
            pltpu.sync_copy(z_hbm.at[pl.ds(0, NTAIL)],
                            acc_sh.at[pl.ds(NSUB * NPSA, NTAIL)])

        plsc.subcore_barrier()

        def body(m_v, di_v):
            pltpu.sync_copy(m_v, acc_sh.at[di_v.at[0]], add=True)

        pltpu.emit_pipeline(
            body,
            grid=(NCHUNK,),
            in_specs=[pl.BlockSpec((CHUNK, D), lambda i: (i, 0)),
                      pl.BlockSpec((1, CHUNK), lambda i: (0, i))],
            out_specs=[],
            core_axis_name=("c", "s"),
            dimension_semantics=(pltpu.PARALLEL,),
        )(m_hbm, di_hbm)

        plsc.subcore_barrier()
        pltpu.sync_copy(acc_sh.at[pl.ds(si * NPSA, NPSA)],
                        out_hbm.at[ci, pl.ds(si * NPSA, NPSA)])

        @pl.when(si == 0)
        def _():
            pltpu.sync_copy(acc_sh.at[pl.ds(NSUB * NPSA, NTAIL)],
                            out_hbm.at[ci, pl.ds(NSUB * NPSA, NTAIL)])

    return k(m, dst.reshape(1, E), zeros)


def kernel(positions, velocities, edge_index, params):
    p = params
    src = edge_index[0].astype(jnp.int32)
    dst = edge_index[1].astype(jnp.int32)

    pv = jnp.concatenate(
        [positions, velocities, jnp.zeros((N, 10), jnp.float32)], axis=1)
    pvs, pvd = _sc_gather2(pv, pv, src, dst, tc_tiling=False)
    ef_in = _ef_in(pvs, pvd)

    ns = jnp.concatenate([positions, velocities], axis=-1)
    x = _encoder(ns, p)
    x = jax.lax.optimization_barrier((x, pvs))[0]

    for lp in p['mp']:
        pi, pj = _tables(x, lp)
        gi, gj = _sc_gather2(pi, pj, dst, src)
        m = _edge_stage(ef_in, gi, gj, p, lp)
        parts = _sc_scatter_add(m, dst)
        x = _node_stage(parts[0], parts[1], x, lp)

    return _decoder(x, p)


# speedup vs baseline: 1.0457x; 1.0457x over previous
import functools

import jax
import jax.numpy as jnp
from jax.experimental import pallas as pl
from jax.experimental.pallas import tpu as pltpu
from jax.experimental.pallas import tpu_sc as plsc

N = 10000
E = 320000
HD = 128
EHD = 64
BE = 3200
EH = E // 2
CHUNK = 128
NSUB = 16
NPSA = 624
NTAIL = N - NSUB * NPSA

_SC_MESH = plsc.VectorSubcoreMesh(core_axis_name="c", subcore_axis_name="s")


def _ln(x, g, b):
    m = jnp.mean(x, axis=-1, keepdims=True)
    v = jnp.mean((x - m) * (x - m), axis=-1, keepdims=True)
    return (x - m) / jnp.sqrt(v + 1e-5) * g + b


def _dot(a, b):
    return jnp.dot(a.astype(jnp.bfloat16), b.astype(jnp.bfloat16),
                   preferred_element_type=jnp.float32)


def _ef_in_body(pvs_ref, pvd_ref, o_ref):
    a = pvs_ref[...]
    b = pvd_ref[...]
    rel_pos = a[:, 0:3] - b[:, 0:3]
    rel_vel = a[:, 3:6] - b[:, 3:6]
    dist = jnp.sqrt(jnp.sum(rel_pos * rel_pos, axis=-1, keepdims=True))
    speed = jnp.sqrt(jnp.sum(rel_vel * rel_vel, axis=-1, keepdims=True))
    o_ref[...] = jnp.concatenate([rel_pos, dist, rel_vel, speed], axis=-1)


def _ef_in(pvs, pvd):
    return pl.pallas_call(
        _ef_in_body,
        grid=(E // BE,),
        in_specs=[pl.BlockSpec((BE, 128), lambda i: (i, 0)),
                  pl.BlockSpec((BE, 128), lambda i: (i, 0))],
        out_specs=pl.BlockSpec((BE, 8), lambda i: (i, 0)),
        out_shape=jax.ShapeDtypeStruct((E, 8), jnp.float32),
    )(pvs, pvd)


def _enc_body(ns_ref, w1_ref, b1_ref, g_ref, bb_ref, w2_ref, b2_ref, o_ref):
    h = _dot(ns_ref[...], w1_ref[...]) + b1_ref[...]
    h = jnp.maximum(_ln(h, g_ref[...], bb_ref[...]), 0.0)
    o_ref[...] = _dot(h, w2_ref[...]) + b2_ref[...]


def _encoder(ns, p):
    return pl.pallas_call(
        _enc_body,
        out_shape=jax.ShapeDtypeStruct((N, HD), jnp.float32),
    )(ns, p['ne_w1'], p['ne_b1'], p['ne_ln_g'], p['ne_ln_b'], p['ne_w2'], p['ne_b2'])


def _tables(x, lp):
    wi = lp['msg_w1'][EHD:EHD + HD]
    wj = lp['msg_w1'][EHD + HD:]

    def body(x_ref, wi_ref, wj_ref, b1_ref, pi_ref, pj_ref):
        x = x_ref[...]
        pi_ref[...] = _dot(x, wi_ref[...]) + b1_ref[...].reshape(1, HD)
        pj_ref[...] = _dot(x, wj_ref[...])

    return pl.pallas_call(
        body,
        out_shape=(jax.ShapeDtypeStruct((N, HD), jnp.float32),
                   jax.ShapeDtypeStruct((N, HD), jnp.float32)),
    )(x, wi, wj, lp['msg_b1'])


def _edge_body(ef_ref, gi_ref, gj_ref, w1_ref, b1_ref, w2_ref, b2_ref,
               we_ref, g_ref, b_ref, mw2_ref, mb2_ref, o_ref):
    t = jnp.maximum(_dot(ef_ref[...], w1_ref[...]) + b1_ref[...], 0.0)
    e = _dot(t, w2_ref[...]) + b2_ref[...]
    m1 = _dot(e, we_ref[...]) + gi_ref[...] + gj_ref[...]
    h = jnp.maximum(_ln(m1, g_ref[...], b_ref[...]), 0.0)
    o_ref[...] = _dot(h, mw2_ref[...]) + mb2_ref[...]


def _edge_stage(ef_in, gi, gj, p, lp):
    we = lp['msg_w1'][0:EHD]
    ne = ef_in.shape[0]
    return pl.pallas_call(
        _edge_body,
        grid=(ne // BE,),
        in_specs=[pl.BlockSpec((BE, 8), lambda i: (i, 0)),
                  pl.BlockSpec((BE, HD), lambda i: (i, 0)),
                  pl.BlockSpec((BE, HD), lambda i: (i, 0)),
                  pl.BlockSpec((8, EHD), lambda i: (0, 0)),
                  pl.BlockSpec((EHD,), lambda i: (0,)),
                  pl.BlockSpec((EHD, EHD), lambda i: (0, 0)),
                  pl.BlockSpec((EHD,), lambda i: (0,)),
                  pl.BlockSpec((EHD, HD), lambda i: (0, 0)),
                  pl.BlockSpec((HD,), lambda i: (0,)),
                  pl.BlockSpec((HD,), lambda i: (0,)),
                  pl.BlockSpec((HD, HD), lambda i: (0, 0)),
                  pl.BlockSpec((HD,), lambda i: (0,))],
        out_specs=pl.BlockSpec((BE, HD), lambda i: (i, 0)),
        out_shape=jax.ShapeDtypeStruct((ne, HD), jnp.float32),
    )(ef_in, gi, gj, p['ef_w1'], p['ef_b1'], p['ef_w2'], p['ef_b2'],
      we, lp['msg_ln_g'], lp['msg_ln_b'], lp['msg_w2'], lp['msg_b2'])


def _node_body(pa_ref, pb_ref, pc_ref, pd_ref, x_ref,
               uw1a_ref, uw1b_ref, ub1_ref, ug_ref, ubb_ref, uw2_ref, ub2_ref,
               o_ref):
    aggr = (pa_ref[...] + pb_ref[...]) + (pc_ref[...] + pd_ref[...])
    x = x_ref[...]
    ui = (_dot(x, uw1a_ref[...]) + _dot(aggr, uw1b_ref[...]) + ub1_ref[...])
    u = jnp.maximum(_ln(ui, ug_ref[...], ubb_ref[...]), 0.0)
    o_ref[...] = _dot(u, uw2_ref[...]) + ub2_ref[...] + x


def _node_stage(pa, pb, pc, pd, x, lp):
    return pl.pallas_call(
        _node_body,
        out_shape=jax.ShapeDtypeStruct((N, HD), jnp.float32),
    )(pa, pb, pc, pd, x,
      lp['upd_w1'][0:HD], lp['upd_w1'][HD:], lp['upd_b1'],
      lp['upd_ln_g'], lp['upd_ln_b'], lp['upd_w2'], lp['upd_b2'])


def _dec_body(x_ref, w1_ref, b1_ref, g_ref, bb_ref, w2_ref, b2_ref, w3_ref,
              b3_ref, o_ref):
    d = _dot(x_ref[...], w1_ref[...]) + b1_ref[...]
    d = jnp.maximum(_ln(d, g_ref[...], bb_ref[...]), 0.0)
    d = jnp.maximum(_dot(d, w2_ref[...]) + b2_ref[...], 0.0)
    o_ref[...] = _dot(d, w3_ref[...]) + b3_ref[...]


def _decoder(x, p):
    return pl.pallas_call(
        _dec_body,
        out_shape=jax.ShapeDtypeStruct((N, 3), jnp.float32),
    )(x, p['dec_w1'], p['dec_b1'], p['dec_ln_g'], p['dec_ln_b'],
      p['dec_w2'], p['dec_b2'], p['dec_w3'], p['dec_b3'])


def _sc_gather2(ti, tj, idx_i, idx_j, tc_tiling=True):
    D = ti.shape[1]
    ne = idx_i.shape[0]

    @functools.partial(
        pl.kernel,
        out_type=(jax.ShapeDtypeStruct((ne, D), jnp.float32),
                  jax.ShapeDtypeStruct((ne, D), jnp.float32)),
        mesh=_SC_MESH,
        compiler_params=pltpu.CompilerParams(use_tc_tiling_on_sc=tc_tiling),
    )
    def k(ti_hbm, tj_hbm, ii_hbm, ij_hbm, gi_hbm, gj_hbm):
        def body(ii_v, ij_v, gi_v, gj_v):
            pltpu.sync_copy(ti_hbm.at[ii_v.at[0]], gi_v)
            pltpu.sync_copy(tj_hbm.at[ij_v.at[0]], gj_v)

        pltpu.emit_pipeline(
            body,
            grid=(ne // CHUNK,),
            in_specs=[pl.BlockSpec((1, CHUNK), lambda i: (0, i)),
                      pl.BlockSpec((1, CHUNK), lambda i: (0, i))],
            out_specs=[pl.BlockSpec((CHUNK, D), lambda i: (i, 0)),
                       pl.BlockSpec((CHUNK, D), lambda i: (i, 0))],
            core_axis_name=("c", "s"),
            dimension_semantics=(pltpu.PARALLEL,),
        )(ii_hbm, ij_hbm, gi_hbm, gj_hbm)

    return k(ti, tj, idx_i.reshape(1, ne), idx_j.reshape(1, ne))


def _sc_scatter_add(m, dst):
    D = m.shape[1]
    ne = m.shape[0]
    zeros = jnp.zeros((NPSA, D), jnp.float32)

    @functools.partial(
        pl.kernel,
        out_type=jax.ShapeDtypeStruct((2, N, D), jnp.float32),
        mesh=_SC_MESH,
        scratch_types=[pltpu.VMEM_SHARED((N, D), jnp.float32)],
    )
    def k(m_hbm, di_hbm, z_hbm, out_hbm, acc_sh):
        ci = jax.lax.axis_index("c")
        si = jax.lax.axis_index("s")
        pltpu.sync_copy(z_hbm, acc_sh.at[pl.ds(si * NPSA, NPSA)])

        @pl.when(si == 0)
        def _():
            pltpu.sync_copy(z_hbm.at[pl.ds(0, NTAIL)],
                            acc_sh.at[pl.ds(NSUB * NPSA, NTAIL)])

        plsc.subcore_barrier()

        def body(m_v, di_v):
            pltpu.sync_copy(m_v, acc_sh.at[di_v.at[0]], add=True)

        pltpu.emit_pipeline(
            body,
            grid=(ne // CHUNK,),
            in_specs=[pl.BlockSpec((CHUNK, D), lambda i: (i, 0)),
                      pl.BlockSpec((1, CHUNK), lambda i: (0, i))],
            out_specs=[],
            core_axis_name=("c", "s"),
            dimension_semantics=(pltpu.PARALLEL,),
        )(m_hbm, di_hbm)

        plsc.subcore_barrier()
        pltpu.sync_copy(acc_sh.at[pl.ds(si * NPSA, NPSA)],
                        out_hbm.at[ci, pl.ds(si * NPSA, NPSA)])

        @pl.when(si == 0)
        def _():
            pltpu.sync_copy(acc_sh.at[pl.ds(NSUB * NPSA, NTAIL)],
                            out_hbm.at[ci, pl.ds(NSUB * NPSA, NTAIL)])

    return k(m, dst.reshape(1, ne), zeros)


def kernel(positions, velocities, edge_index, params):
    p = params
    src = edge_index[0].astype(jnp.int32)
    dst = edge_index[1].astype(jnp.int32)

    pv = jnp.concatenate(
        [positions, velocities, jnp.zeros((N, 122), jnp.float32)], axis=1)
    pvs, pvd = _sc_gather2(pv, pv, src, dst)
    ef_in = _ef_in(pvs, pvd)

    ns = jnp.concatenate([positions, velocities], axis=-1)
    x = _encoder(ns, p)
    x = jax.lax.optimization_barrier((x, pvs))[0]

    dstA, dstB = dst[:EH], dst[EH:]
    srcA, srcB = src[:EH], src[EH:]
    efA = jax.lax.slice(ef_in, (0, 0), (EH, 8))
    efB = jax.lax.slice(ef_in, (EH, 0), (E, 8))

    for lp in p['mp']:
        pi, pj = _tables(x, lp)
        giA, gjA = _sc_gather2(pi, pj, dstA, srcA)
        dstB_d = jax.lax.optimization_barrier((dstB, giA[0]))[0]
        giB, gjB = _sc_gather2(pi, pj, dstB_d, srcB)
        mA = _edge_stage(efA, giA, gjA, p, lp)
        mA_d = jax.lax.optimization_barrier((mA, giB[0]))[0]
        partsA = _sc_scatter_add(mA_d, dstA)
        mB = _edge_stage(efB, giB, gjB, p, lp)
        mB_d = jax.lax.optimization_barrier((mB, partsA[0, 0]))[0]
        partsB = _sc_scatter_add(mB_d, dstB)
        x = _node_stage(partsA[0], partsA[1], partsB[0], partsB[1], x, lp)

    return _decoder(x, p)
